# baseline (speedup 1.0000x reference)
import functools

import jax
import jax.numpy as jnp
from jax import lax
from jax.experimental import pallas as pl
from jax.experimental.pallas import tpu as pltpu
from jax.experimental.pallas import tpu_sc as plsc

_CP = 128
_HB = 16


def _pack_pair_table(xb):
    C, H, W = xb.shape
    nblk = H // _HB
    rows = (nblk + 1) * _HB

    def body(pr_ref, cu_ref, o_ref):
        cur = cu_ref[...].reshape(C, _HB * W).T
        prev_last = pr_ref[:, _HB - 1, :].T
        top = jnp.concatenate([prev_last, cur[: (_HB - 1) * W]], axis=0)
        topw = lax.bitcast_convert_type(
            jnp.pad(top.astype(jnp.bfloat16), ((0, 0), (0, _CP - C))),
            jnp.uint16).astype(jnp.uint32)
        botw = lax.bitcast_convert_type(
            jnp.pad(cur.astype(jnp.bfloat16), ((0, 0), (0, _CP - C))),
            jnp.uint16).astype(jnp.uint32)
        word = topw | (botw << 16)
        o_ref[...] = lax.bitcast_convert_type(word, jnp.int32).reshape(
            _HB, W, _CP)

    out = pl.pallas_call(
        body,
        grid=(nblk + 1,),
        in_specs=[
            pl.BlockSpec((C, _HB, W),
                         lambda h: (0, jnp.clip(h - 1, 0, nblk - 1), 0)),
            pl.BlockSpec((C, _HB, W),
                         lambda h: (0, jnp.minimum(h, nblk - 1), 0)),
        ],
        out_specs=pl.BlockSpec((_HB, W, _CP), lambda h: (h, 0, 0)),
        out_shape=jax.ShapeDtypeStruct((rows, W, _CP), jnp.int32),
    )(xb, xb)
    return out.reshape(rows * W, _CP)


def _sc_sample(xt, posx, posy, C, H, W, NP):
    info = plsc.get_sparse_core_info()
    NC, NS, L = info.num_cores, info.num_subcores, info.num_lanes
    NW = NC * NS
    PPW = NP // NW
    G = 80
    NCH = PPW // G
    CB = C // L

    mesh = plsc.VectorSubcoreMesh(core_axis_name="c", subcore_axis_name="s")

    @functools.partial(
        pl.kernel, mesh=mesh,
        out_type=jax.ShapeDtypeStruct((NP, C), jnp.float32),
        scratch_types=[
            pltpu.VMEM((PPW,), jnp.float32),
            pltpu.VMEM((PPW,), jnp.float32),
            [pltpu.VMEM((4, G), jnp.float32)] * 2,
            [[pltpu.VMEM((G, _CP), jnp.int32)] * 2] * 2,
            pltpu.VMEM((G, C), jnp.float32),
            [[pltpu.SemaphoreType.DMA] * 2] * 2,
        ],
    )
    def body(xt_hbm, px_hbm, py_hbm, out_hbm, px_v, py_v,
             wb2, rows2, ov, sem2):
        wid = lax.axis_index("c") * NS + lax.axis_index("s")
        base = wid * PPW
        pltpu.sync_copy(px_hbm.at[pl.ds(base, PPW)], px_v)
        pltpu.sync_copy(py_hbm.at[pl.ds(base, PPW)], py_v)

        def calc_fire(g, bi):
            wb = wb2[bi]
            goff = g * G
            for j in range(G // L):
                off = goff + j * L
                px = px_v[pl.ds(off, L)]
                py = py_v[pl.ds(off, L)]
                gx = 2.0 * (px / float(W - 1)) - 1.0
                gy = 2.0 * (py / float(H - 1)) - 1.0
                ix = ((gx + 1.0) * float(W) - 1.0) / 2.0
                iy = ((gy + 1.0) * float(H) - 1.0) / 2.0
                x0i = (ix + 1.0).astype(jnp.int32) - 1
                y0i = (iy + 1.0).astype(jnp.int32) - 1
                x0f = x0i.astype(jnp.float32)
                y0f = y0i.astype(jnp.float32)
                wx1 = ix - x0f
                wx0 = 1.0 - wx1
                wy1 = iy - y0f
                wy0 = 1.0 - wy1
                vx0 = jnp.clip(x0f + 1.0, 0.0, 1.0)
                vx1 = jnp.clip(float(W) - (x0f + 1.0), 0.0, 1.0)
                vy0 = jnp.clip(y0f + 1.0, 0.0, 1.0)
                vy1 = jnp.clip(float(H) - (y0f + 1.0), 0.0, 1.0)
                sl = pl.ds(j * L, L)
                wb[0, sl] = wx0 * wy0 * (vx0 * vy0)
                wb[1, sl] = wx1 * wy0 * (vx1 * vy0)
                wb[2, sl] = wx0 * wy1 * (vx0 * vy1)
                wb[3, sl] = wx1 * wy1 * (vx1 * vy1)
                xc0 = jnp.clip(x0i, 0, W - 1)
                xc1 = jnp.clip(x0i + 1, 0, W - 1)
                rbase = (y0i + 1) * W
                pltpu.async_copy(xt_hbm.at[rbase + xc0],
                                 rows2[bi][0].at[sl], sem2[bi][0])
                pltpu.async_copy(xt_hbm.at[rbase + xc1],
                                 rows2[bi][1].at[sl], sem2[bi][1])

        def drain(bi):
            zidx = lax.iota(jnp.int32, L)
            for k in range(2):
                for j in range(G // L):
                    pltpu.make_async_copy(
                        xt_hbm.at[zidx],
                        rows2[bi][k].at[pl.ds(j * L, L)],
                        sem2[bi][k]).wait()

        def combine(g, bi):
            r0, r1 = rows2[bi]
            wb = wb2[bi]

            def gbody(j, carry2):
                gsl = pl.ds(j * L, L)
                w00v = wb[0, gsl]
                w10v = wb[1, gsl]
                w01v = wb[2, gsl]
                w11v = wb[3, gsl]
                for t in range(L):
                    p = j * L + t
                    w0 = w00v[t]
                    w1 = w10v[t]
                    w2 = w01v[t]
                    w3 = w11v[t]
                    himask = jnp.full((L,), -65536, jnp.int32)
                    for cb in range(CB):
                        csl = pl.ds(cb * L, L)
                        v0 = r0[p, csl]
                        v1 = r1[p, csl]
                        a0 = lax.bitcast_convert_type(v0 << 16, jnp.float32)
                        b0 = lax.bitcast_convert_type(v0 & himask, jnp.float32)
                        a1 = lax.bitcast_convert_type(v1 << 16, jnp.float32)
                        b1 = lax.bitcast_convert_type(v1 & himask, jnp.float32)
                        ov[p, csl] = a0 * w0 + a1 * w1 + b0 * w2 + b1 * w3
                return carry2

            lax.fori_loop(0, G // L, gbody, 0)
            pltpu.sync_copy(ov, out_hbm.at[pl.ds(base + g * G, G)])

        calc_fire(0, 0)

        def pair(g2, carry):
            g0 = g2 * 2
            calc_fire(g0 + 1, 1)
            drain(0)
            combine(g0, 0)

            @pl.when(g2 < NCH // 2 - 1)
            def _():
                calc_fire(g0 + 2, 0)

            drain(1)
            combine(g0 + 1, 1)
            return carry

        lax.fori_loop(0, NCH // 2, pair, 0)

    return body(xt, posx, posy)


def kernel(x, pos, H, W):
    B, C, Hs, Ws = x.shape
    _, N, _ = pos.shape
    NW, G = 32, 80
    NPb = ((N + NW * G - 1) // (NW * G)) * (NW * G)

    posx = jnp.pad(pos[..., 0], ((0, 0), (0, NPb - N)))
    posy = jnp.pad(pos[..., 1], ((0, 0), (0, NPb - N)))
    outs = []
    for b in range(B):
        xt = _pack_pair_table(x[b])
        ob = _sc_sample(xt, posx[b], posy[b], C, Hs, Ws, NPb)
        outs.append(ob[:N])
    return jnp.stack(outs)


# speedup vs baseline: 1.2024x; 1.2024x over previous
import functools

import jax
import jax.numpy as jnp
from jax import lax
from jax.experimental import pallas as pl
from jax.experimental.pallas import tpu as pltpu
from jax.experimental.pallas import tpu_sc as plsc

_CP = 128
_HB = 16


def _pack_pair_table(x, lastrows, b):
    _, C, H, W = x.shape
    nblk = H // _HB
    rows = (nblk + 1) * _HB

    def body(pr_ref, cu_ref, o_ref):
        cur = cu_ref[0].reshape(C, _HB * W).T
        prev_last = pr_ref[0, 0].T
        top = jnp.concatenate([prev_last, cur[: (_HB - 1) * W]], axis=0)
        topw = lax.bitcast_convert_type(
            jnp.pad(top.astype(jnp.bfloat16), ((0, 0), (0, _CP - C))),
            jnp.uint16).astype(jnp.uint32)
        botw = lax.bitcast_convert_type(
            jnp.pad(cur.astype(jnp.bfloat16), ((0, 0), (0, _CP - C))),
            jnp.uint16).astype(jnp.uint32)
        word = topw | (botw << 16)
        o_ref[...] = lax.bitcast_convert_type(word, jnp.int32).reshape(
            _HB, W, _CP)

    out = pl.pallas_call(
        body,
        grid=(nblk + 1,),
        in_specs=[
            pl.BlockSpec((1, 1, C, W),
                         lambda h: (b, jnp.clip(h - 1, 0, nblk - 1), 0, 0)),
            pl.BlockSpec((1, C, _HB, W),
                         lambda h: (b, 0, jnp.minimum(h, nblk - 1), 0)),
        ],
        out_specs=pl.BlockSpec((_HB, W, _CP), lambda h: (h, 0, 0)),
        out_shape=jax.ShapeDtypeStruct((rows, W, _CP), jnp.int32),
    )(lastrows, x)
    return out.reshape(rows * W, _CP)


def _sc_sample(xt, posx, posy, C, H, W, NP):
    info = plsc.get_sparse_core_info()
    NC, NS, L = info.num_cores, info.num_subcores, info.num_lanes
    NW = NC * NS
    PPW = NP // NW
    G = 80
    NCH = PPW // G
    CB = C // L

    mesh = plsc.VectorSubcoreMesh(core_axis_name="c", subcore_axis_name="s")

    @functools.partial(
        pl.kernel, mesh=mesh,
        out_type=jax.ShapeDtypeStruct((NP, C), jnp.float32),
        scratch_types=[
            pltpu.VMEM((PPW,), jnp.float32),
            pltpu.VMEM((PPW,), jnp.float32),
            [pltpu.VMEM((4, G), jnp.float32)] * 2,
            [[pltpu.VMEM((G, _CP), jnp.int32)] * 2] * 2,
            pltpu.VMEM((G, C), jnp.float32),
            [[pltpu.SemaphoreType.DMA] * 2] * 2,
        ],
    )
    def body(xt_hbm, px_hbm, py_hbm, out_hbm, px_v, py_v,
             wb2, rows2, ov, sem2):
        wid = lax.axis_index("c") * NS + lax.axis_index("s")
        base = wid * PPW
        pltpu.sync_copy(px_hbm.at[pl.ds(base, PPW)], px_v)
        pltpu.sync_copy(py_hbm.at[pl.ds(base, PPW)], py_v)

        def calc_fire(g, bi):
            wb = wb2[bi]
            goff = g * G
            for j in range(G // L):
                off = goff + j * L
                px = px_v[pl.ds(off, L)]
                py = py_v[pl.ds(off, L)]
                gx = 2.0 * (px / float(W - 1)) - 1.0
                gy = 2.0 * (py / float(H - 1)) - 1.0
                ix = ((gx + 1.0) * float(W) - 1.0) / 2.0
                iy = ((gy + 1.0) * float(H) - 1.0) / 2.0
                x0i = (ix + 1.0).astype(jnp.int32) - 1
                y0i = (iy + 1.0).astype(jnp.int32) - 1
                x0f = x0i.astype(jnp.float32)
                y0f = y0i.astype(jnp.float32)
                wx1 = ix - x0f
                wx0 = 1.0 - wx1
                wy1 = iy - y0f
                wy0 = 1.0 - wy1
                vx0 = jnp.clip(x0f + 1.0, 0.0, 1.0)
                vx1 = jnp.clip(float(W) - (x0f + 1.0), 0.0, 1.0)
                vy0 = jnp.clip(y0f + 1.0, 0.0, 1.0)
                vy1 = jnp.clip(float(H) - (y0f + 1.0), 0.0, 1.0)
                sl = pl.ds(j * L, L)
                wb[0, sl] = wx0 * wy0 * (vx0 * vy0)
                wb[1, sl] = wx1 * wy0 * (vx1 * vy0)
                wb[2, sl] = wx0 * wy1 * (vx0 * vy1)
                wb[3, sl] = wx1 * wy1 * (vx1 * vy1)
                xc0 = jnp.clip(x0i, 0, W - 1)
                xc1 = jnp.clip(x0i + 1, 0, W - 1)
                rbase = (y0i + 1) * W
                pltpu.async_copy(xt_hbm.at[rbase + xc0],
                                 rows2[bi][0].at[sl], sem2[bi][0])
                pltpu.async_copy(xt_hbm.at[rbase + xc1],
                                 rows2[bi][1].at[sl], sem2[bi][1])

        def drain(bi):
            zidx = lax.iota(jnp.int32, L)
            for k in range(2):
                for j in range(G // L):
                    pltpu.make_async_copy(
                        xt_hbm.at[zidx],
                        rows2[bi][k].at[pl.ds(j * L, L)],
                        sem2[bi][k]).wait()

        def combine(g, bi):
            r0, r1 = rows2[bi]
            wb = wb2[bi]

            def gbody(j, carry2):
                gsl = pl.ds(j * L, L)
                w00v = wb[0, gsl]
                w10v = wb[1, gsl]
                w01v = wb[2, gsl]
                w11v = wb[3, gsl]
                for t in range(L):
                    p = j * L + t
                    w0 = w00v[t]
                    w1 = w10v[t]
                    w2 = w01v[t]
                    w3 = w11v[t]
                    himask = jnp.full((L,), -65536, jnp.int32)
                    for cb in range(CB):
                        csl = pl.ds(cb * L, L)
                        v0 = r0[p, csl]
                        v1 = r1[p, csl]
                        a0 = lax.bitcast_convert_type(v0 << 16, jnp.float32)
                        b0 = lax.bitcast_convert_type(v0 & himask, jnp.float32)
                        a1 = lax.bitcast_convert_type(v1 << 16, jnp.float32)
                        b1 = lax.bitcast_convert_type(v1 & himask, jnp.float32)
                        ov[p, csl] = a0 * w0 + a1 * w1 + b0 * w2 + b1 * w3
                return carry2

            lax.fori_loop(0, G // L, gbody, 0)
            pltpu.sync_copy(ov, out_hbm.at[pl.ds(base + g * G, G)])

        calc_fire(0, 0)

        def pair(g2, carry):
            g0 = g2 * 2
            calc_fire(g0 + 1, 1)
            drain(0)
            combine(g0, 0)

            @pl.when(g2 < NCH // 2 - 1)
            def _():
                calc_fire(g0 + 2, 0)

            drain(1)
            combine(g0 + 1, 1)
            return carry

        lax.fori_loop(0, NCH // 2, pair, 0)

    return body(xt, posx, posy)


def kernel(x, pos, H, W):
    B, C, Hs, Ws = x.shape
    _, N, _ = pos.shape
    NW, G = 32, 80
    NPb = ((N + NW * G - 1) // (NW * G)) * (NW * G)

    posx = jnp.pad(pos[..., 0], ((0, 0), (0, NPb - N)))
    posy = jnp.pad(pos[..., 1], ((0, 0), (0, NPb - N)))
    lastrows = x[:, :, _HB - 1::_HB, :].transpose(0, 2, 1, 3)
    outs = []
    for b in range(B):
        xt = _pack_pair_table(x, lastrows, b)
        ob = _sc_sample(xt, posx[b], posy[b], C, Hs, Ws, NPb)
        outs.append(ob[:N])
    return jnp.stack(outs)


# speedup vs baseline: 1.4514x; 1.2071x over previous
import functools

import jax
import jax.numpy as jnp
from jax import lax
from jax.experimental import pallas as pl
from jax.experimental.pallas import tpu as pltpu
from jax.experimental.pallas import tpu_sc as plsc

_CP = 128
_HB = 16


def _pack_pair_table(x, b):
    _, C, H, W = x.shape
    nblk = H // _HB

    def body(cu_ref, nx_ref, o_ref):
        cur = cu_ref[0].reshape(C, _HB * W).T
        nxt_first = nx_ref[0, :, 0, :].T
        bot = jnp.concatenate([cur[W:], nxt_first], axis=0)
        topw = lax.bitcast_convert_type(
            jnp.pad(cur.astype(jnp.bfloat16), ((0, 0), (0, _CP - C))),
            jnp.uint16).astype(jnp.uint32)
        botw = lax.bitcast_convert_type(
            jnp.pad(bot.astype(jnp.bfloat16), ((0, 0), (0, _CP - C))),
            jnp.uint16).astype(jnp.uint32)
        word = topw | (botw << 16)
        o_ref[...] = lax.bitcast_convert_type(word, jnp.int32).reshape(
            _HB, W, _CP)

    out = pl.pallas_call(
        body,
        grid=(nblk,),
        in_specs=[
            pl.BlockSpec((1, C, _HB, W), lambda h: (b, 0, h, 0)),
            pl.BlockSpec((1, C, 8, W),
                         lambda h: (b, 0, jnp.minimum(2 * h + 2, H // 8 - 1),
                                    0)),
        ],
        out_specs=pl.BlockSpec((_HB, W, _CP), lambda h: (h, 0, 0)),
        out_shape=jax.ShapeDtypeStruct((H, W, _CP), jnp.int32),
    )(x, x)
    return out.reshape(H * W, _CP)


def _sc_sample(xt, posx, posy, C, H, W, NP):
    info = plsc.get_sparse_core_info()
    NC, NS, L = info.num_cores, info.num_subcores, info.num_lanes
    NW = NC * NS
    PPW = NP // NW
    G = 80
    NCH = PPW // G
    CB = C // L

    mesh = plsc.VectorSubcoreMesh(core_axis_name="c", subcore_axis_name="s")

    @functools.partial(
        pl.kernel, mesh=mesh,
        out_type=jax.ShapeDtypeStruct((NP, C), jnp.float32),
        scratch_types=[
            pltpu.VMEM((PPW,), jnp.float32),
            pltpu.VMEM((PPW,), jnp.float32),
            [pltpu.VMEM((4, G), jnp.float32)] * 2,
            [[pltpu.VMEM((G, _CP), jnp.int32)] * 2] * 2,
            pltpu.VMEM((G, C), jnp.float32),
            [[pltpu.SemaphoreType.DMA] * 2] * 2,
        ],
    )
    def body(xt_hbm, px_hbm, py_hbm, out_hbm, px_v, py_v,
             wb2, rows2, ov, sem2):
        wid = lax.axis_index("c") * NS + lax.axis_index("s")
        base = wid * PPW
        pltpu.sync_copy(px_hbm.at[pl.ds(base, PPW)], px_v)
        pltpu.sync_copy(py_hbm.at[pl.ds(base, PPW)], py_v)

        def calc_fire(g, bi):
            wb = wb2[bi]
            goff = g * G
            for j in range(G // L):
                off = goff + j * L
                px = px_v[pl.ds(off, L)]
                py = py_v[pl.ds(off, L)]
                gx = 2.0 * (px / float(W - 1)) - 1.0
                gy = 2.0 * (py / float(H - 1)) - 1.0
                ix = ((gx + 1.0) * float(W) - 1.0) / 2.0
                iy = ((gy + 1.0) * float(H) - 1.0) / 2.0
                x0i = (ix + 1.0).astype(jnp.int32) - 1
                y0i = (iy + 1.0).astype(jnp.int32) - 1
                x0f = x0i.astype(jnp.float32)
                y0f = y0i.astype(jnp.float32)
                wx1 = ix - x0f
                wx0 = 1.0 - wx1
                wy1 = iy - y0f
                wy0 = 1.0 - wy1
                vx0 = jnp.clip(x0f + 1.0, 0.0, 1.0)
                vx1 = jnp.clip(float(W) - (x0f + 1.0), 0.0, 1.0)
                vy0 = jnp.clip(y0f + 1.0, 0.0, 1.0)
                vy1 = jnp.clip(float(H) - (y0f + 1.0), 0.0, 1.0)
                w00 = wx0 * wy0 * (vx0 * vy0)
                w10 = wx1 * wy0 * (vx1 * vy0)
                w01 = wx0 * wy1 * (vx0 * vy1)
                w11 = wx1 * wy1 * (vx1 * vy1)
                ind = jnp.clip(-y0f, 0.0, 1.0)
                sl = pl.ds(j * L, L)
                wb[0, sl] = w00 + w01 * ind
                wb[1, sl] = w10 + w11 * ind
                wb[2, sl] = w01 * (1.0 - ind)
                wb[3, sl] = w11 * (1.0 - ind)
                xc0 = jnp.clip(x0i, 0, W - 1)
                xc1 = jnp.clip(x0i + 1, 0, W - 1)
                rbase = (y0i + ind.astype(jnp.int32)) * W
                pltpu.async_copy(xt_hbm.at[rbase + xc0],
                                 rows2[bi][0].at[sl], sem2[bi][0])
                pltpu.async_copy(xt_hbm.at[rbase + xc1],
                                 rows2[bi][1].at[sl], sem2[bi][1])

        def drain(bi):
            zidx = lax.iota(jnp.int32, L)
            for k in range(2):
                for j in range(G // L):
                    pltpu.make_async_copy(
                        xt_hbm.at[zidx],
                        rows2[bi][k].at[pl.ds(j * L, L)],
                        sem2[bi][k]).wait()

        def combine(g, bi):
            r0, r1 = rows2[bi]
            wb = wb2[bi]

            def gbody(j, carry2):
                gsl = pl.ds(j * L, L)
                w00v = wb[0, gsl]
                w10v = wb[1, gsl]
                w01v = wb[2, gsl]
                w11v = wb[3, gsl]
                for t in range(L):
                    p = j * L + t
                    w0 = w00v[t]
                    w1 = w10v[t]
                    w2 = w01v[t]
                    w3 = w11v[t]
                    himask = jnp.full((L,), -65536, jnp.int32)
                    for cb in range(CB):
                        csl = pl.ds(cb * L, L)
                        v0 = r0[p, csl]
                        v1 = r1[p, csl]
                        a0 = lax.bitcast_convert_type(v0 << 16, jnp.float32)
                        b0 = lax.bitcast_convert_type(v0 & himask, jnp.float32)
                        a1 = lax.bitcast_convert_type(v1 << 16, jnp.float32)
                        b1 = lax.bitcast_convert_type(v1 & himask, jnp.float32)
                        ov[p, csl] = a0 * w0 + a1 * w1 + b0 * w2 + b1 * w3
                return carry2

            lax.fori_loop(0, G // L, gbody, 0)
            pltpu.sync_copy(ov, out_hbm.at[pl.ds(base + g * G, G)])

        calc_fire(0, 0)

        def pair(g2, carry):
            g0 = g2 * 2
            calc_fire(g0 + 1, 1)
            drain(0)
            combine(g0, 0)

            @pl.when(g2 < NCH // 2 - 1)
            def _():
                calc_fire(g0 + 2, 0)

            drain(1)
            combine(g0 + 1, 1)
            return carry

        lax.fori_loop(0, NCH // 2, pair, 0)

    return body(xt, posx, posy)


def kernel(x, pos, H, W):
    B, C, Hs, Ws = x.shape
    _, N, _ = pos.shape
    NW, G = 32, 80
    NPb = ((N + NW * G - 1) // (NW * G)) * (NW * G)

    posx = jnp.pad(pos[..., 0], ((0, 0), (0, NPb - N)))
    posy = jnp.pad(pos[..., 1], ((0, 0), (0, NPb - N)))
    outs = []
    for b in range(B):
        xt = _pack_pair_table(x, b)
        ob = _sc_sample(xt, posx[b], posy[b], C, Hs, Ws, NPb)
        outs.append(ob[:N])
    return jnp.stack(outs)


# speedup vs baseline: 1.5948x; 1.0988x over previous
import functools

import jax
import jax.numpy as jnp
from jax import lax
from jax.experimental import pallas as pl
from jax.experimental.pallas import tpu as pltpu
from jax.experimental.pallas import tpu_sc as plsc

_CP = 128
_HB = 48


def _pack_pair_table(x, b):
    _, C, H, W = x.shape
    nblk = H // _HB

    def body(cu_ref, nx_ref, o_ref):
        cur = cu_ref[0].reshape(C, _HB * W).T
        nxt_first = nx_ref[0, :, 0, :].T
        bot = jnp.concatenate([cur[W:], nxt_first], axis=0)
        topw = lax.bitcast_convert_type(
            jnp.pad(cur.astype(jnp.bfloat16), ((0, 0), (0, _CP - C))),
            jnp.uint16).astype(jnp.uint32)
        botw = lax.bitcast_convert_type(
            jnp.pad(bot.astype(jnp.bfloat16), ((0, 0), (0, _CP - C))),
            jnp.uint16).astype(jnp.uint32)
        word = topw | (botw << 16)
        o_ref[...] = lax.bitcast_convert_type(word, jnp.int32).reshape(
            _HB, W, _CP)

    out = pl.pallas_call(
        body,
        grid=(nblk,),
        in_specs=[
            pl.BlockSpec((1, C, _HB, W), lambda h: (b, 0, h, 0)),
            pl.BlockSpec((1, C, 8, W),
                         lambda h: (b, 0,
                                    jnp.minimum((h + 1) * (_HB // 8),
                                                H // 8 - 1), 0)),
        ],
        out_specs=pl.BlockSpec((_HB, W, _CP), lambda h: (h, 0, 0)),
        out_shape=jax.ShapeDtypeStruct((H, W, _CP), jnp.int32),
    )(x, x)
    return out.reshape(H * W, _CP)


def _sc_sample(xt, posx, posy, C, H, W, NP):
    info = plsc.get_sparse_core_info()
    NC, NS, L = info.num_cores, info.num_subcores, info.num_lanes
    NW = NC * NS
    PPW = NP // NW
    G = 80
    NCH = PPW // G
    CB = C // L

    mesh = plsc.VectorSubcoreMesh(core_axis_name="c", subcore_axis_name="s")

    @functools.partial(
        pl.kernel, mesh=mesh,
        out_type=jax.ShapeDtypeStruct((NP, C), jnp.float32),
        scratch_types=[
            pltpu.VMEM((PPW,), jnp.float32),
            pltpu.VMEM((PPW,), jnp.float32),
            [pltpu.VMEM((4, G), jnp.float32)] * 2,
            [[pltpu.VMEM((G, _CP), jnp.int32)] * 2] * 2,
            pltpu.VMEM((G, C), jnp.float32),
            [[pltpu.SemaphoreType.DMA] * 2] * 2,
        ],
    )
    def body(xt_hbm, px_hbm, py_hbm, out_hbm, px_v, py_v,
             wb2, rows2, ov, sem2):
        wid = lax.axis_index("c") * NS + lax.axis_index("s")
        base = wid * PPW
        pltpu.sync_copy(px_hbm.at[pl.ds(base, PPW)], px_v)
        pltpu.sync_copy(py_hbm.at[pl.ds(base, PPW)], py_v)

        def calc_fire(g, bi):
            wb = wb2[bi]
            goff = g * G
            for j in range(G // L):
                off = goff + j * L
                px = px_v[pl.ds(off, L)]
                py = py_v[pl.ds(off, L)]
                gx = 2.0 * (px / float(W - 1)) - 1.0
                gy = 2.0 * (py / float(H - 1)) - 1.0
                ix = ((gx + 1.0) * float(W) - 1.0) / 2.0
                iy = ((gy + 1.0) * float(H) - 1.0) / 2.0
                x0i = (ix + 1.0).astype(jnp.int32) - 1
                y0i = (iy + 1.0).astype(jnp.int32) - 1
                x0f = x0i.astype(jnp.float32)
                y0f = y0i.astype(jnp.float32)
                wx1 = ix - x0f
                wx0 = 1.0 - wx1
                wy1 = iy - y0f
                wy0 = 1.0 - wy1
                vx0 = jnp.clip(x0f + 1.0, 0.0, 1.0)
                vx1 = jnp.clip(float(W) - (x0f + 1.0), 0.0, 1.0)
                vy0 = jnp.clip(y0f + 1.0, 0.0, 1.0)
                vy1 = jnp.clip(float(H) - (y0f + 1.0), 0.0, 1.0)
                w00 = wx0 * wy0 * (vx0 * vy0)
                w10 = wx1 * wy0 * (vx1 * vy0)
                w01 = wx0 * wy1 * (vx0 * vy1)
                w11 = wx1 * wy1 * (vx1 * vy1)
                ind = jnp.clip(-y0f, 0.0, 1.0)
                sl = pl.ds(j * L, L)
                wb[0, sl] = w00 + w01 * ind
                wb[1, sl] = w10 + w11 * ind
                wb[2, sl] = w01 * (1.0 - ind)
                wb[3, sl] = w11 * (1.0 - ind)
                xc0 = jnp.clip(x0i, 0, W - 1)
                xc1 = jnp.clip(x0i + 1, 0, W - 1)
                rbase = (y0i + ind.astype(jnp.int32)) * W
                pltpu.async_copy(xt_hbm.at[rbase + xc0],
                                 rows2[bi][0].at[sl], sem2[bi][0])
                pltpu.async_copy(xt_hbm.at[rbase + xc1],
                                 rows2[bi][1].at[sl], sem2[bi][1])

        def drain(bi):
            zidx = lax.iota(jnp.int32, L)
            for k in range(2):
                for j in range(G // L):
                    pltpu.make_async_copy(
                        xt_hbm.at[zidx],
                        rows2[bi][k].at[pl.ds(j * L, L)],
                        sem2[bi][k]).wait()

        def combine(g, bi):
            r0, r1 = rows2[bi]
            wb = wb2[bi]

            def gbody(j, carry2):
                gsl = pl.ds(j * L, L)
                w00v = wb[0, gsl]
                w10v = wb[1, gsl]
                w01v = wb[2, gsl]
                w11v = wb[3, gsl]
                for t in range(L):
                    p = j * L + t
                    w0 = w00v[t]
                    w1 = w10v[t]
                    w2 = w01v[t]
                    w3 = w11v[t]
                    himask = jnp.full((L,), -65536, jnp.int32)
                    for cb in range(CB):
                        csl = pl.ds(cb * L, L)
                        v0 = r0[p, csl]
                        v1 = r1[p, csl]
                        a0 = lax.bitcast_convert_type(v0 << 16, jnp.float32)
                        b0 = lax.bitcast_convert_type(v0 & himask, jnp.float32)
                        a1 = lax.bitcast_convert_type(v1 << 16, jnp.float32)
                        b1 = lax.bitcast_convert_type(v1 & himask, jnp.float32)
                        ov[p, csl] = a0 * w0 + a1 * w1 + b0 * w2 + b1 * w3
                return carry2

            lax.fori_loop(0, G // L, gbody, 0)
            pltpu.sync_copy(ov, out_hbm.at[pl.ds(base + g * G, G)])

        calc_fire(0, 0)

        def pair(g2, carry):
            g0 = g2 * 2
            calc_fire(g0 + 1, 1)
            drain(0)
            combine(g0, 0)

            @pl.when(g2 < NCH // 2 - 1)
            def _():
                calc_fire(g0 + 2, 0)

            drain(1)
            combine(g0 + 1, 1)
            return carry

        lax.fori_loop(0, NCH // 2, pair, 0)

    return body(xt, posx, posy)


def kernel(x, pos, H, W):
    B, C, Hs, Ws = x.shape
    _, N, _ = pos.shape
    NW, G = 32, 80
    NPb = ((N + NW * G - 1) // (NW * G)) * (NW * G)

    posx = jnp.pad(pos[..., 0], ((0, 0), (0, NPb - N)))
    posy = jnp.pad(pos[..., 1], ((0, 0), (0, NPb - N)))
    outs = []
    for b in range(B):
        xt = _pack_pair_table(x, b)
        ob = _sc_sample(xt, posx[b], posy[b], C, Hs, Ws, NPb)
        outs.append(ob[:N])
    return jnp.stack(outs)


# speedup vs baseline: 1.6096x; 1.0092x over previous
import functools

import jax
import jax.numpy as jnp
from jax import lax
from jax.experimental import pallas as pl
from jax.experimental.pallas import tpu as pltpu
from jax.experimental.pallas import tpu_sc as plsc

_CP = 128
_HB = 64


def _pack_pair_table(x, b):
    _, C, H, W = x.shape
    nblk = H // _HB

    def body(cu_ref, nx_ref, o_ref):
        cur = cu_ref[0].reshape(C, _HB * W).T
        nxt_first = nx_ref[0, :, 0, :].T
        bot = jnp.concatenate([cur[W:], nxt_first], axis=0)
        topw = lax.bitcast_convert_type(
            jnp.pad(cur.astype(jnp.bfloat16), ((0, 0), (0, _CP - C))),
            jnp.uint16).astype(jnp.uint32)
        botw = lax.bitcast_convert_type(
            jnp.pad(bot.astype(jnp.bfloat16), ((0, 0), (0, _CP - C))),
            jnp.uint16).astype(jnp.uint32)
        word = topw | (botw << 16)
        o_ref[...] = lax.bitcast_convert_type(word, jnp.int32).reshape(
            _HB, W, _CP)

    out = pl.pallas_call(
        body,
        grid=(nblk,),
        in_specs=[
            pl.BlockSpec((1, C, _HB, W), lambda h: (b, 0, h, 0)),
            pl.BlockSpec((1, C, 8, W),
                         lambda h: (b, 0,
                                    jnp.minimum((h + 1) * (_HB // 8),
                                                H // 8 - 1), 0)),
        ],
        out_specs=pl.BlockSpec((_HB, W, _CP), lambda h: (h, 0, 0)),
        out_shape=jax.ShapeDtypeStruct((H, W, _CP), jnp.int32),
    )(x, x)
    return out.reshape(H * W, _CP)


def _sc_sample(xt, posx, posy, C, H, W, NP):
    info = plsc.get_sparse_core_info()
    NC, NS, L = info.num_cores, info.num_subcores, info.num_lanes
    NW = NC * NS
    PPW = NP // NW
    G = 80
    NCH = PPW // G
    CB = C // L

    mesh = plsc.VectorSubcoreMesh(core_axis_name="c", subcore_axis_name="s")

    @functools.partial(
        pl.kernel, mesh=mesh,
        out_type=jax.ShapeDtypeStruct((NP, C), jnp.float32),
        scratch_types=[
            pltpu.VMEM((PPW,), jnp.float32),
            pltpu.VMEM((PPW,), jnp.float32),
            [pltpu.VMEM((4, G), jnp.float32)] * 2,
            [[pltpu.VMEM((G, _CP), jnp.int32)] * 2] * 2,
            pltpu.VMEM((G, C), jnp.float32),
            [[pltpu.SemaphoreType.DMA] * 2] * 2,
        ],
    )
    def body(xt_hbm, px_hbm, py_hbm, out_hbm, px_v, py_v,
             wb2, rows2, ov, sem2):
        wid = lax.axis_index("c") * NS + lax.axis_index("s")
        base = wid * PPW
        pltpu.sync_copy(px_hbm.at[pl.ds(base, PPW)], px_v)
        pltpu.sync_copy(py_hbm.at[pl.ds(base, PPW)], py_v)

        def calc_fire(g, bi):
            wb = wb2[bi]
            goff = g * G
            for j in range(G // L):
                off = goff + j * L
                px = px_v[pl.ds(off, L)]
                py = py_v[pl.ds(off, L)]
                gx = 2.0 * (px / float(W - 1)) - 1.0
                gy = 2.0 * (py / float(H - 1)) - 1.0
                ix = ((gx + 1.0) * float(W) - 1.0) / 2.0
                iy = ((gy + 1.0) * float(H) - 1.0) / 2.0
                x0i = (ix + 1.0).astype(jnp.int32) - 1
                y0i = (iy + 1.0).astype(jnp.int32) - 1
                x0f = x0i.astype(jnp.float32)
                y0f = y0i.astype(jnp.float32)
                wx1 = ix - x0f
                wx0 = 1.0 - wx1
                wy1 = iy - y0f
                wy0 = 1.0 - wy1
                vx0 = jnp.clip(x0f + 1.0, 0.0, 1.0)
                vx1 = jnp.clip(float(W) - (x0f + 1.0), 0.0, 1.0)
                vy0 = jnp.clip(y0f + 1.0, 0.0, 1.0)
                vy1 = jnp.clip(float(H) - (y0f + 1.0), 0.0, 1.0)
                w00 = wx0 * wy0 * (vx0 * vy0)
                w10 = wx1 * wy0 * (vx1 * vy0)
                w01 = wx0 * wy1 * (vx0 * vy1)
                w11 = wx1 * wy1 * (vx1 * vy1)
                ind = jnp.clip(-y0f, 0.0, 1.0)
                sl = pl.ds(j * L, L)
                wb[0, sl] = w00 + w01 * ind
                wb[1, sl] = w10 + w11 * ind
                wb[2, sl] = w01 * (1.0 - ind)
                wb[3, sl] = w11 * (1.0 - ind)
                xc0 = jnp.clip(x0i, 0, W - 1)
                xc1 = jnp.clip(x0i + 1, 0, W - 1)
                rbase = (y0i + ind.astype(jnp.int32)) * W
                pltpu.async_copy(xt_hbm.at[rbase + xc0],
                                 rows2[bi][0].at[sl], sem2[bi][0])
                pltpu.async_copy(xt_hbm.at[rbase + xc1],
                                 rows2[bi][1].at[sl], sem2[bi][1])

        def drain(bi):
            zidx = lax.iota(jnp.int32, L)
            for k in range(2):
                for j in range(G // L):
                    pltpu.make_async_copy(
                        xt_hbm.at[zidx],
                        rows2[bi][k].at[pl.ds(j * L, L)],
                        sem2[bi][k]).wait()

        def combine(g, bi):
            r0, r1 = rows2[bi]
            wb = wb2[bi]

            def gbody(j, carry2):
                gsl = pl.ds(j * L, L)
                w00v = wb[0, gsl]
                w10v = wb[1, gsl]
                w01v = wb[2, gsl]
                w11v = wb[3, gsl]
                for t in range(L):
                    p = j * L + t
                    w0 = w00v[t]
                    w1 = w10v[t]
                    w2 = w01v[t]
                    w3 = w11v[t]
                    himask = jnp.full((L,), -65536, jnp.int32)
                    for cb in range(CB):
                        csl = pl.ds(cb * L, L)
                        v0 = r0[p, csl]
                        v1 = r1[p, csl]
                        a0 = lax.bitcast_convert_type(v0 << 16, jnp.float32)
                        b0 = lax.bitcast_convert_type(v0 & himask, jnp.float32)
                        a1 = lax.bitcast_convert_type(v1 << 16, jnp.float32)
                        b1 = lax.bitcast_convert_type(v1 & himask, jnp.float32)
                        ov[p, csl] = a0 * w0 + a1 * w1 + b0 * w2 + b1 * w3
                return carry2

            lax.fori_loop(0, G // L, gbody, 0)
            pltpu.sync_copy(ov, out_hbm.at[pl.ds(base + g * G, G)])

        calc_fire(0, 0)

        def pair(g2, carry):
            g0 = g2 * 2
            calc_fire(g0 + 1, 1)
            drain(0)
            combine(g0, 0)

            @pl.when(g2 < NCH // 2 - 1)
            def _():
                calc_fire(g0 + 2, 0)

            drain(1)
            combine(g0 + 1, 1)
            return carry

        lax.fori_loop(0, NCH // 2, pair, 0)

    return body(xt, posx, posy)


def kernel(x, pos, H, W):
    B, C, Hs, Ws = x.shape
    _, N, _ = pos.shape
    NW, G = 32, 80
    NPb = ((N + NW * G - 1) // (NW * G)) * (NW * G)

    posx = jnp.pad(pos[..., 0], ((0, 0), (0, NPb - N)))
    posy = jnp.pad(pos[..., 1], ((0, 0), (0, NPb - N)))
    outs = []
    for b in range(B):
        xt = _pack_pair_table(x, b)
        ob = _sc_sample(xt, posx[b], posy[b], C, Hs, Ws, NPb)
        outs.append(ob[:N])
    return jnp.stack(outs)
